# baseline (speedup 1.0000x reference)
import functools

import jax
import jax.numpy as jnp
from jax import lax
from jax.experimental import pallas as pl
from jax.experimental.pallas import tpu as pltpu
from jax.experimental.pallas import tpu_sc as plsc

NUM_ITEMS = 100000
EMBED_DIM = 64
BATCH = 16384
PAD_DIM = 128

_NC = 2
_NS = 16
_NW = _NC * _NS
_B_PER_W = BATCH // _NW
_CHUNK = 128
_NCH = _B_PER_W // _CHUNK

_mesh = plsc.VectorSubcoreMesh(core_axis_name="c", subcore_axis_name="s")


@functools.partial(
    pl.kernel,
    mesh=_mesh,
    out_type=jax.ShapeDtypeStruct((BATCH, PAD_DIM), jnp.float32),
    scratch_types=[
        pltpu.VMEM((_B_PER_W,), jnp.int32),
        pltpu.VMEM((_B_PER_W, EMBED_DIM), jnp.float32),
        pltpu.SemaphoreType.DMA,
    ],
    compiler_params=pltpu.CompilerParams(use_tc_tiling_on_sc=False),
)
def _gather_kernel(idx_hbm, table_hbm, out_hbm, idx_v, rows_v, sem):
    wid = lax.axis_index("s") * _NC + lax.axis_index("c")
    base = wid * _B_PER_W
    pltpu.sync_copy(idx_hbm.at[pl.ds(base, _B_PER_W)], idx_v)
    copies = [
        pltpu.async_copy(
            table_hbm.at[idx_v.at[pl.ds(j * _CHUNK, _CHUNK)]],
            rows_v.at[pl.ds(j * _CHUNK, _CHUNK)],
            sem,
        )
        for j in range(_NCH)
    ]
    for c in copies:
        c.wait()
    pltpu.sync_copy(
        rows_v, out_hbm.at[pl.ds(base, _B_PER_W), pl.ds(0, EMBED_DIM)]
    )


_TC_ROWS = 2048


def _strip_pad_body(x_ref, o_ref):
    o_ref[...] = x_ref[:, :EMBED_DIM]


_strip_pad = pl.pallas_call(
    _strip_pad_body,
    grid=(BATCH // _TC_ROWS,),
    in_specs=[pl.BlockSpec((_TC_ROWS, PAD_DIM), lambda i: (i, 0))],
    out_specs=pl.BlockSpec((_TC_ROWS, EMBED_DIM), lambda i: (i, 0)),
    out_shape=jax.ShapeDtypeStruct((BATCH, EMBED_DIM), jnp.float32),
)


def kernel(item_indices, embedding_table):
    padded = _gather_kernel(item_indices.astype(jnp.int32), embedding_table)
    return _strip_pad(padded)


# speedup vs baseline: 1.0117x; 1.0117x over previous
import functools

import jax
import jax.numpy as jnp
from jax import lax
from jax.experimental import pallas as pl
from jax.experimental.pallas import tpu as pltpu
from jax.experimental.pallas import tpu_sc as plsc

NUM_ITEMS = 100000
EMBED_DIM = 64
BATCH = 16384
PAD_DIM = 128
WIDE_ROWS = NUM_ITEMS * EMBED_DIM // PAD_DIM

_NC = 2
_NS = 16
_NW = _NC * _NS
_B_PER_W = BATCH // _NW
_CHUNK = 128
_NCH = _B_PER_W // _CHUNK
_L = 16

_mesh = plsc.VectorSubcoreMesh(core_axis_name="c", subcore_axis_name="s")


@functools.partial(
    pl.kernel,
    mesh=_mesh,
    out_type=jax.ShapeDtypeStruct((BATCH, PAD_DIM), jnp.float32),
    scratch_types=[
        pltpu.VMEM((_B_PER_W,), jnp.int32),
        pltpu.VMEM((_B_PER_W,), jnp.int32),
        pltpu.VMEM((_NCH, _CHUNK, PAD_DIM), jnp.float32),
        pltpu.VMEM((2, _CHUNK, PAD_DIM), jnp.float32),
        pltpu.SemaphoreType.DMA,
        pltpu.SemaphoreType.DMA,
    ],
    compiler_params=pltpu.CompilerParams(
        use_tc_tiling_on_sc=True, needs_layout_passes=False
    ),
)
def _gather_kernel(idx_hbm, table_hbm, out_hbm, idx_v, jdx_v, wide_v, rows_v,
                   sem, osem):
    wid = lax.axis_index("s") * _NC + lax.axis_index("c")
    base = wid * _B_PER_W
    pltpu.sync_copy(idx_hbm.at[pl.ds(base, _B_PER_W)], idx_v)
    for g in range(_B_PER_W // _L):
        iv = idx_v[pl.ds(g * _L, _L)]
        jdx_v[pl.ds(g * _L, _L)] = lax.shift_right_logical(iv, 1)
    copies = [
        pltpu.async_copy(
            table_hbm.at[jdx_v.at[pl.ds(j * _CHUNK, _CHUNK)]],
            wide_v.at[j],
            sem,
        )
        for j in range(_NCH)
    ]
    iota = lax.iota(jnp.int32, _L)
    out_copies = []
    for j in range(_NCH):
        copies[j].wait()
        if j >= 2:
            out_copies[j - 2].wait()

        def _extract_group(g, _, j=j):
            k0 = j * _CHUNK + g * _L
            iv = idx_v[pl.ds(k0, _L)]
            h16 = lax.mul(lax.rem(iv, 2), EMBED_DIM)
            for t in range(_L):
                rv = jnp.zeros((_L,), jnp.int32) + (g * _L + t)
                col0 = h16[t] + iota
                for c0 in range(0, EMBED_DIM, _L):
                    v = plsc.load_gather(wide_v.at[j], [rv, col0 + c0])
                    plsc.store_scatter(rows_v.at[j % 2], [rv, iota + c0], v)
            return _

        lax.fori_loop(0, _CHUNK // _L, _extract_group, 0)
        out_copies.append(
            pltpu.async_copy(
                rows_v.at[j % 2],
                out_hbm.at[pl.ds(base + j * _CHUNK, _CHUNK)],
                osem,
            )
        )
    for c in out_copies[-2:]:
        c.wait()


def kernel(item_indices, embedding_table):
    wide = jnp.reshape(embedding_table, (WIDE_ROWS, PAD_DIM))
    padded = _gather_kernel(item_indices.astype(jnp.int32), wide)
    return padded[:, :EMBED_DIM]


# speedup vs baseline: 1.1468x; 1.1336x over previous
import functools

import jax
import jax.numpy as jnp
from jax import lax
from jax.experimental import pallas as pl
from jax.experimental.pallas import tpu as pltpu
from jax.experimental.pallas import tpu_sc as plsc

NUM_ITEMS = 100000
EMBED_DIM = 64
BATCH = 16384
PAD_DIM = 128
TILE_ROWS = 8
N_TILES = NUM_ITEMS // TILE_ROWS

_NC = 2
_NS = 16
_NW = _NC * _NS
_B_PER_W = BATCH // _NW
_L = 16
_OCH = 64
_NOCH = _B_PER_W // _OCH
_NG = _OCH // _L

_mesh = plsc.VectorSubcoreMesh(core_axis_name="c", subcore_axis_name="s")


@functools.partial(
    pl.kernel,
    mesh=_mesh,
    out_type=jax.ShapeDtypeStruct((BATCH, PAD_DIM), jnp.float32),
    scratch_types=[
        pltpu.VMEM((_B_PER_W + _L,), jnp.int32),
        pltpu.VMEM((_L, TILE_ROWS, EMBED_DIM), jnp.float32),
        pltpu.VMEM((2, _OCH, PAD_DIM), jnp.float32),
        pltpu.SemaphoreType.DMA,
        pltpu.SemaphoreType.DMA,
    ],
    compiler_params=pltpu.CompilerParams(
        use_tc_tiling_on_sc=True, needs_layout_passes=False
    ),
)
def _gather_kernel(idx_hbm, table_hbm, out_hbm, idx_v, ring_v, rows_v,
                   sem, osem):
    wid = lax.axis_index("s") * _NC + lax.axis_index("c")
    base = wid * _B_PER_W
    pltpu.sync_copy(idx_hbm.at[pl.ds(base, _B_PER_W)], idx_v.at[pl.ds(0, _B_PER_W)])

    tv0 = lax.shift_right_logical(idx_v[pl.ds(0, _L)], 3)
    for t in range(_L):
        pltpu.async_copy(table_hbm.at[tv0[t]], ring_v.at[t], sem)

    def _group(g, carry):
        iv = idx_v[pl.ds(g * _L, _L)]
        rv = lax.rem(iv, TILE_ROWS)
        tvn = lax.shift_right_logical(idx_v[pl.ds((g + 1) * _L, _L)], 3)
        for t in range(_L):
            pltpu.make_async_copy(
                table_hbm.at[0], ring_v.at[t], sem
            ).wait()
            r = rv[t]
            for c0 in range(0, EMBED_DIM, _L):
                rows_v[carry, lax.rem(g, _NG) * _L + t, pl.ds(c0, _L)] = (
                    ring_v[t, r, pl.ds(c0, _L)]
                )

            @pl.when(g * _L + t + _L < _B_PER_W)
            def _refill():
                pltpu.async_copy(table_hbm.at[tvn[t]], ring_v.at[t], sem)

        return carry

    for j in range(_NOCH):
        if j >= 2:
            pltpu.make_async_copy(
                rows_v.at[j % 2], out_hbm.at[pl.ds(0, _OCH)], osem
            ).wait()
        lax.fori_loop(j * _NG, (j + 1) * _NG, _group, j % 2, unroll=False)
        pltpu.async_copy(
            rows_v.at[j % 2],
            out_hbm.at[pl.ds(base + j * _OCH, _OCH)],
            osem,
        )
    for _ in range(2):
        pltpu.make_async_copy(
            rows_v.at[0], out_hbm.at[pl.ds(0, _OCH)], osem
        ).wait()


def kernel(item_indices, embedding_table):
    tiled = jnp.reshape(embedding_table, (N_TILES, TILE_ROWS, EMBED_DIM))
    padded = _gather_kernel(item_indices.astype(jnp.int32), tiled)
    return padded[:, :EMBED_DIM]


# speedup vs baseline: 1.4866x; 1.2963x over previous
import functools

import jax
import jax.numpy as jnp
from jax import lax
from jax.experimental import pallas as pl
from jax.experimental.pallas import tpu as pltpu
from jax.experimental.pallas import tpu_sc as plsc

NUM_ITEMS = 100000
EMBED_DIM = 64
BATCH = 16384
PAD_DIM = 128
TILE_ROWS = 8
N_TILES = NUM_ITEMS // TILE_ROWS

_NC = 2
_NS = 16
_NW = _NC * _NS
_B_PER_W = BATCH // _NW
_L = 16
_OCH = 64
_NOCH = _B_PER_W // _OCH
_NG = _OCH // _L

_mesh = plsc.VectorSubcoreMesh(core_axis_name="c", subcore_axis_name="s")


@functools.partial(
    pl.kernel,
    mesh=_mesh,
    out_type=jax.ShapeDtypeStruct((BATCH, PAD_DIM), jnp.float32),
    scratch_types=[
        pltpu.VMEM((_B_PER_W + _L,), jnp.int32),
        pltpu.VMEM((_L, TILE_ROWS, EMBED_DIM), jnp.float32),
        pltpu.VMEM((2, _OCH, PAD_DIM), jnp.float32),
        pltpu.SemaphoreType.DMA,
        pltpu.SemaphoreType.DMA,
    ],
    compiler_params=pltpu.CompilerParams(
        use_tc_tiling_on_sc=True, needs_layout_passes=False
    ),
)
def _gather_kernel(idx_hbm, table_hbm, out_hbm, idx_v, ring_v, rows_v,
                   sem, osem):
    wid = lax.axis_index("s") * _NC + lax.axis_index("c")
    base = wid * _B_PER_W
    pltpu.sync_copy(idx_hbm.at[pl.ds(base, _B_PER_W)], idx_v.at[pl.ds(0, _B_PER_W)])

    def _group(g, carry):
        iv = idx_v[pl.ds(g * _L, _L)]
        tv = lax.shift_right_logical(iv, 3)
        rv = lax.rem(iv, TILE_ROWS)
        for t in range(_L):
            pltpu.async_copy(
                table_hbm.at[tv[t], rv[t]],
                rows_v.at[carry, lax.rem(g, _NG) * _L + t, pl.ds(0, EMBED_DIM)],
                sem,
            )
        return carry

    for j in range(_NOCH):
        if j >= 2:
            pltpu.make_async_copy(
                rows_v.at[j % 2], out_hbm.at[pl.ds(0, _OCH)], osem
            ).wait()
        lax.fori_loop(j * _NG, (j + 1) * _NG, _group, j % 2, unroll=False)
        for _g in range(_NG * _L):
            pltpu.make_async_copy(
                table_hbm.at[0, 0],
                rows_v.at[0, 0, pl.ds(0, EMBED_DIM)],
                sem,
            ).wait()
        pltpu.async_copy(
            rows_v.at[j % 2],
            out_hbm.at[pl.ds(base + j * _OCH, _OCH)],
            osem,
        )
    for _ in range(2):
        pltpu.make_async_copy(
            rows_v.at[0], out_hbm.at[pl.ds(0, _OCH)], osem
        ).wait()


def kernel(item_indices, embedding_table):
    tiled = jnp.reshape(embedding_table, (N_TILES, TILE_ROWS, EMBED_DIM))
    padded = _gather_kernel(item_indices.astype(jnp.int32), tiled)
    return padded[:, :EMBED_DIM]


# speedup vs baseline: 1.6572x; 1.1148x over previous
import functools

import jax
import jax.numpy as jnp
from jax import lax
from jax.experimental import pallas as pl
from jax.experimental.pallas import tpu as pltpu
from jax.experimental.pallas import tpu_sc as plsc

NUM_ITEMS = 100000
EMBED_DIM = 64
BATCH = 16384
PAD_DIM = 128
TILE_ROWS = 8
N_TILES = NUM_ITEMS // TILE_ROWS

_NC = 2
_NS = 16
_NW = _NC * _NS
_B_PER_W = BATCH // _NW
_L = 16
_OCH = 64
_NOCH = _B_PER_W // _OCH
_NG = _OCH // _L

_mesh = plsc.VectorSubcoreMesh(core_axis_name="c", subcore_axis_name="s")


@functools.partial(
    pl.kernel,
    mesh=_mesh,
    out_type=jax.ShapeDtypeStruct((BATCH, PAD_DIM), jnp.float32),
    scratch_types=[
        pltpu.VMEM((_B_PER_W,), jnp.int32),
        pltpu.VMEM((2, _OCH, PAD_DIM), jnp.float32),
        pltpu.VMEM((_OCH * EMBED_DIM,), jnp.int32),
        pltpu.SemaphoreType.DMA,
        pltpu.SemaphoreType.DMA,
        pltpu.SemaphoreType.DMA,
    ],
    compiler_params=pltpu.CompilerParams(
        use_tc_tiling_on_sc=True, needs_layout_passes=False
    ),
)
def _gather_kernel(idx_hbm, table_hbm, out_hbm, idx_v, rows_v, dummy_v,
                   semA, semB, osem):
    wid = lax.axis_index("s") * _NC + lax.axis_index("c")
    base = wid * _B_PER_W
    pltpu.sync_copy(idx_hbm.at[pl.ds(base, _B_PER_W)], idx_v)
    sems = (semA, semB)

    def _fire_chunk(j):
        sem = sems[j % 2]
        buf = j % 2

        def _group(g, carry):
            iv = idx_v[pl.ds(g * _L, _L)]
            tv = lax.shift_right_logical(iv, 3)
            rv = lax.rem(iv, TILE_ROWS)
            for t in range(_L):
                pltpu.async_copy(
                    table_hbm.at[tv[t], rv[t]],
                    rows_v.at[buf, lax.rem(g, _NG) * _L + t,
                              pl.ds(0, EMBED_DIM)],
                    sem,
                )
            return carry

        lax.fori_loop(j * _NG, (j + 1) * _NG, _group, 0, unroll=False)

    def _drain_chunk(j):
        pltpu.make_async_copy(
            idx_hbm.at[pl.ds(0, _OCH * EMBED_DIM)], dummy_v, sems[j % 2]
        ).wait()

    _fire_chunk(0)
    for j in range(_NOCH):
        if j + 1 < _NOCH:
            if j >= 1:
                pltpu.make_async_copy(
                    rows_v.at[0], out_hbm.at[pl.ds(0, _OCH)], osem
                ).wait()
            _fire_chunk(j + 1)
        _drain_chunk(j)
        pltpu.async_copy(
            rows_v.at[j % 2],
            out_hbm.at[pl.ds(base + j * _OCH, _OCH)],
            osem,
        )
    for _ in range(2):
        pltpu.make_async_copy(
            rows_v.at[0], out_hbm.at[pl.ds(0, _OCH)], osem
        ).wait()


def kernel(item_indices, embedding_table):
    tiled = jnp.reshape(embedding_table, (N_TILES, TILE_ROWS, EMBED_DIM))
    padded = _gather_kernel(item_indices.astype(jnp.int32), tiled)
    return padded[:, :EMBED_DIM]


# speedup vs baseline: 1.7523x; 1.0574x over previous
import functools

import jax
import jax.numpy as jnp
from jax import lax
from jax.experimental import pallas as pl
from jax.experimental.pallas import tpu as pltpu
from jax.experimental.pallas import tpu_sc as plsc

NUM_ITEMS = 100000
EMBED_DIM = 64
BATCH = 16384
PAD_DIM = 128
TILE_ROWS = 8
N_TILES = NUM_ITEMS // TILE_ROWS

_NC = 2
_NS = 16
_NW = _NC * _NS
_B_PER_W = BATCH // _NW
_L = 16
_OCH = 128
_NOCH = _B_PER_W // _OCH
_NG = _OCH // _L

_mesh = plsc.VectorSubcoreMesh(core_axis_name="c", subcore_axis_name="s")


@functools.partial(
    pl.kernel,
    mesh=_mesh,
    out_type=jax.ShapeDtypeStruct((BATCH, PAD_DIM), jnp.float32),
    scratch_types=[
        pltpu.VMEM((_B_PER_W,), jnp.int32),
        pltpu.VMEM((2, _OCH, PAD_DIM), jnp.float32),
        pltpu.VMEM((_OCH * EMBED_DIM,), jnp.int32),
        pltpu.SemaphoreType.DMA,
        pltpu.SemaphoreType.DMA,
        pltpu.SemaphoreType.DMA,
    ],
    compiler_params=pltpu.CompilerParams(
        use_tc_tiling_on_sc=True, needs_layout_passes=False
    ),
)
def _gather_kernel(idx_hbm, table_hbm, out_hbm, idx_v, rows_v, dummy_v,
                   semA, semB, osem):
    wid = lax.axis_index("s") * _NC + lax.axis_index("c")
    base = wid * _B_PER_W
    pltpu.sync_copy(idx_hbm.at[pl.ds(base, _B_PER_W)], idx_v)
    sems = (semA, semB)

    def _fire_chunk(j):
        sem = sems[j % 2]
        buf = j % 2

        def _group(g, carry):
            iv = idx_v[pl.ds(g * _L, _L)]
            tv = lax.shift_right_logical(iv, 3)
            rv = lax.rem(iv, TILE_ROWS)
            for t in range(_L):
                pltpu.async_copy(
                    table_hbm.at[tv[t], rv[t]],
                    rows_v.at[buf, lax.rem(g, _NG) * _L + t,
                              pl.ds(0, EMBED_DIM)],
                    sem,
                )
            return carry

        lax.fori_loop(j * _NG, (j + 1) * _NG, _group, 0, unroll=False)

    def _drain_chunk(j):
        pltpu.make_async_copy(
            idx_hbm.at[pl.ds(0, _OCH * EMBED_DIM)], dummy_v, sems[j % 2]
        ).wait()

    _fire_chunk(0)
    for j in range(_NOCH):
        if j + 1 < _NOCH:
            if j >= 1:
                pltpu.make_async_copy(
                    rows_v.at[0], out_hbm.at[pl.ds(0, _OCH)], osem
                ).wait()
            _fire_chunk(j + 1)
        _drain_chunk(j)
        pltpu.async_copy(
            rows_v.at[j % 2],
            out_hbm.at[pl.ds(base + j * _OCH, _OCH)],
            osem,
        )
    for _ in range(2):
        pltpu.make_async_copy(
            rows_v.at[0], out_hbm.at[pl.ds(0, _OCH)], osem
        ).wait()


def kernel(item_indices, embedding_table):
    tiled = jnp.reshape(embedding_table, (N_TILES, TILE_ROWS, EMBED_DIM))
    padded = _gather_kernel(item_indices.astype(jnp.int32), tiled)
    return padded[:, :EMBED_DIM]


# speedup vs baseline: 1.7908x; 1.0220x over previous
import functools

import jax
import jax.numpy as jnp
from jax import lax
from jax.experimental import pallas as pl
from jax.experimental.pallas import tpu as pltpu
from jax.experimental.pallas import tpu_sc as plsc

NUM_ITEMS = 100000
EMBED_DIM = 64
BATCH = 16384
PAD_DIM = 128
TILE_ROWS = 8
N_TILES = NUM_ITEMS // TILE_ROWS

_NC = 2
_NS = 16
_NW = _NC * _NS
_B_PER_W = BATCH // _NW
_L = 16
_OCH = 256
_NOCH = _B_PER_W // _OCH
_NG = _OCH // _L

_mesh = plsc.VectorSubcoreMesh(core_axis_name="c", subcore_axis_name="s")


@functools.partial(
    pl.kernel,
    mesh=_mesh,
    out_type=jax.ShapeDtypeStruct((BATCH, PAD_DIM), jnp.float32),
    scratch_types=[
        pltpu.VMEM((_B_PER_W,), jnp.int32),
        pltpu.VMEM((2, _OCH, PAD_DIM), jnp.float32),
        pltpu.VMEM((_OCH * EMBED_DIM,), jnp.int32),
        pltpu.SemaphoreType.DMA,
        pltpu.SemaphoreType.DMA,
        pltpu.SemaphoreType.DMA,
    ],
    compiler_params=pltpu.CompilerParams(
        use_tc_tiling_on_sc=True, needs_layout_passes=False
    ),
)
def _gather_kernel(idx_hbm, table_hbm, out_hbm, idx_v, rows_v, dummy_v,
                   semA, semB, osem):
    wid = lax.axis_index("s") * _NC + lax.axis_index("c")
    base = wid * _B_PER_W
    pltpu.sync_copy(idx_hbm.at[pl.ds(base, _B_PER_W)], idx_v)
    sems = (semA, semB)

    def _fire_chunk(j):
        sem = sems[j % 2]
        buf = j % 2

        def _group(g, carry):
            iv = idx_v[pl.ds(g * _L, _L)]
            tv = lax.shift_right_logical(iv, 3)
            rv = lax.rem(iv, TILE_ROWS)
            for t in range(_L):
                pltpu.async_copy(
                    table_hbm.at[tv[t], rv[t]],
                    rows_v.at[buf, lax.rem(g, _NG) * _L + t,
                              pl.ds(0, EMBED_DIM)],
                    sem,
                )
            return carry

        lax.fori_loop(j * _NG, (j + 1) * _NG, _group, 0, unroll=False)

    def _drain_chunk(j):
        pltpu.make_async_copy(
            idx_hbm.at[pl.ds(0, _OCH * EMBED_DIM)], dummy_v, sems[j % 2]
        ).wait()

    _fire_chunk(0)
    for j in range(_NOCH):
        if j + 1 < _NOCH:
            if j >= 1:
                pltpu.make_async_copy(
                    rows_v.at[0], out_hbm.at[pl.ds(0, _OCH)], osem
                ).wait()
            _fire_chunk(j + 1)
        _drain_chunk(j)
        pltpu.async_copy(
            rows_v.at[j % 2],
            out_hbm.at[pl.ds(base + j * _OCH, _OCH)],
            osem,
        )
    for _ in range(2):
        pltpu.make_async_copy(
            rows_v.at[0], out_hbm.at[pl.ds(0, _OCH)], osem
        ).wait()


def kernel(item_indices, embedding_table):
    tiled = jnp.reshape(embedding_table, (N_TILES, TILE_ROWS, EMBED_DIM))
    padded = _gather_kernel(item_indices.astype(jnp.int32), tiled)
    return padded[:, :EMBED_DIM]
